# baseline (speedup 1.0000x reference)
import jax
import jax.numpy as jnp
from jax.experimental import pallas as pl
from jax.experimental.pallas import tpu as pltpu


def _vq_body(z_ref, cb_ref, x_ref, idx_ref, cbn_ref):
    @pl.when(jnp.logical_and(pl.program_id(0) == 0, pl.program_id(1) == 0))
    def _():
        cb0 = cb_ref[...]
        cbn_ref[...] = 0.5 * jnp.sum(cb0 * cb0, axis=1, keepdims=True)

    zb = z_ref[0]
    cb = cb_ref[...]
    scores = jax.lax.dot_general(
        cb, zb, (((1,), (0,)), ((), ())),
        preferred_element_type=jnp.float32)
    rank = scores - cbn_ref[...]
    idx = jnp.argmax(rank, axis=0).astype(jnp.int32)
    K = cb.shape[0]
    T = zb.shape[1]
    onehot = (jax.lax.broadcasted_iota(jnp.int32, (K, T), 0)
              == idx[None, :]).astype(jnp.float32)
    xv = jax.lax.dot_general(
        onehot, cb, (((0,), (0,)), ((), ())),
        preferred_element_type=jnp.float32)
    x_ref[0] = xv
    idx_ref[0, 0, 0] = idx


def kernel(z, codebook):
    B, D, H, W = z.shape
    HW = H * W
    K = codebook.shape[0]
    z3 = z.reshape(B, D, HW)
    T = min(512, HW)
    NT = HW // T
    x, idx = pl.pallas_call(
        _vq_body,
        grid=(B, NT),
        in_specs=[
            pl.BlockSpec((1, D, T), lambda b, t: (b, 0, t)),
            pl.BlockSpec((K, D), lambda b, t: (0, 0)),
        ],
        out_specs=[
            pl.BlockSpec((1, T, D), lambda b, t: (b, t, 0)),
            pl.BlockSpec((1, 1, 1, T), lambda b, t: (b, t, 0, 0)),
        ],
        out_shape=[
            jax.ShapeDtypeStruct((B, HW, D), jnp.float32),
            jax.ShapeDtypeStruct((B, NT, 1, T), jnp.int32),
        ],
        scratch_shapes=[pltpu.VMEM((K, 1), jnp.float32)],
    )(z3, codebook)
    return x, idx.reshape(B, HW)


# speedup vs baseline: 1.3366x; 1.3366x over previous
import jax
import jax.numpy as jnp
from jax.experimental import pallas as pl
from jax.experimental.pallas import tpu as pltpu


def _vq_body(z_ref, cb_ref, x_ref, idx_ref, cbn_ref):
    @pl.when(jnp.logical_and(pl.program_id(0) == 0, pl.program_id(1) == 0))
    def _():
        cb0 = cb_ref[...]
        cbn_ref[...] = 0.5 * jnp.sum(cb0 * cb0, axis=1, keepdims=True)

    zb = z_ref[0]
    cb = cb_ref[...]
    scores = jax.lax.dot_general(
        cb, zb, (((1,), (0,)), ((), ())),
        preferred_element_type=jnp.float32)
    rank = scores - cbn_ref[...]
    idx = jnp.argmax(rank, axis=0).astype(jnp.int32)
    K = cb.shape[0]
    T = zb.shape[1]
    onehot = (jax.lax.broadcasted_iota(jnp.int32, (K, T), 0)
              == idx[None, :]).astype(jnp.float32)
    xv = jax.lax.dot_general(
        onehot, cb, (((0,), (0,)), ((), ())),
        preferred_element_type=jnp.float32)
    x_ref[0] = xv
    idx_ref[0, 0, 0] = idx


def kernel(z, codebook):
    B, D, H, W = z.shape
    HW = H * W
    K = codebook.shape[0]
    z3 = z.reshape(B, D, HW)
    T = min(1024, HW)
    NT = HW // T
    x, idx = pl.pallas_call(
        _vq_body,
        grid=(B, NT),
        in_specs=[
            pl.BlockSpec((1, D, T), lambda b, t: (b, 0, t)),
            pl.BlockSpec((K, D), lambda b, t: (0, 0)),
        ],
        out_specs=[
            pl.BlockSpec((1, T, D), lambda b, t: (b, t, 0)),
            pl.BlockSpec((1, 1, 1, T), lambda b, t: (b, t, 0, 0)),
        ],
        out_shape=[
            jax.ShapeDtypeStruct((B, HW, D), jnp.float32),
            jax.ShapeDtypeStruct((B, NT, 1, T), jnp.int32),
        ],
        scratch_shapes=[pltpu.VMEM((K, 1), jnp.float32)],
    )(z3, codebook)
    return x, idx.reshape(B, HW)


# speedup vs baseline: 1.5235x; 1.1398x over previous
import jax
import jax.numpy as jnp
from jax.experimental import pallas as pl
from jax.experimental.pallas import tpu as pltpu


def _vq_body(z_ref, cb_ref, x_ref, idx_ref, cbn_ref):
    @pl.when(jnp.logical_and(pl.program_id(0) == 0, pl.program_id(1) == 0))
    def _():
        cb0 = cb_ref[...]
        cbn_ref[...] = 0.5 * jnp.sum(cb0 * cb0, axis=1, keepdims=True)

    zb = z_ref[0]
    cb = cb_ref[...]
    scores = jax.lax.dot_general(
        cb, zb, (((1,), (0,)), ((), ())),
        preferred_element_type=jnp.float32)
    rank = scores - cbn_ref[...]
    idx = jnp.argmax(rank, axis=0).astype(jnp.int32)
    K = cb.shape[0]
    T = zb.shape[1]
    onehot = (jax.lax.broadcasted_iota(jnp.int32, (K, T), 0)
              == idx[None, :]).astype(jnp.float32)
    xv = jax.lax.dot_general(
        onehot, cb, (((0,), (0,)), ((), ())),
        preferred_element_type=jnp.float32)
    x_ref[0] = xv
    idx_ref[0, 0, 0] = idx


def kernel(z, codebook):
    B, D, H, W = z.shape
    HW = H * W
    K = codebook.shape[0]
    z3 = z.reshape(B, D, HW)
    T = min(2048, HW)
    NT = HW // T
    x, idx = pl.pallas_call(
        _vq_body,
        grid=(B, NT),
        in_specs=[
            pl.BlockSpec((1, D, T), lambda b, t: (b, 0, t)),
            pl.BlockSpec((K, D), lambda b, t: (0, 0)),
        ],
        out_specs=[
            pl.BlockSpec((1, T, D), lambda b, t: (b, t, 0)),
            pl.BlockSpec((1, 1, 1, T), lambda b, t: (b, t, 0, 0)),
        ],
        out_shape=[
            jax.ShapeDtypeStruct((B, HW, D), jnp.float32),
            jax.ShapeDtypeStruct((B, NT, 1, T), jnp.int32),
        ],
        scratch_shapes=[pltpu.VMEM((K, 1), jnp.float32)],
    )(z3, codebook)
    return x, idx.reshape(B, HW)


# speedup vs baseline: 1.6129x; 1.0587x over previous
import jax
import jax.numpy as jnp
from jax.experimental import pallas as pl
from jax.experimental.pallas import tpu as pltpu


def _vq_body(z_ref, cb_ref, x_ref, idx_ref, cbn_ref):
    @pl.when(jnp.logical_and(pl.program_id(0) == 0, pl.program_id(1) == 0))
    def _():
        cb0 = cb_ref[...]
        cbn_ref[...] = 0.5 * jnp.sum(cb0 * cb0, axis=1, keepdims=True)

    zb = z_ref[0]
    cb = cb_ref[...]
    scores = jax.lax.dot_general(
        cb, zb, (((1,), (0,)), ((), ())),
        preferred_element_type=jnp.float32)
    rank = scores - cbn_ref[...]
    idx = jnp.argmax(rank, axis=0).astype(jnp.int32)
    K = cb.shape[0]
    T = zb.shape[1]
    onehot = (jax.lax.broadcasted_iota(jnp.int32, (K, T), 0)
              == idx[None, :]).astype(jnp.float32)
    xv = jax.lax.dot_general(
        onehot, cb, (((0,), (0,)), ((), ())),
        preferred_element_type=jnp.float32)
    x_ref[0] = xv
    idx_ref[0, 0, 0] = idx


def kernel(z, codebook):
    B, D, H, W = z.shape
    HW = H * W
    K = codebook.shape[0]
    z3 = z.reshape(B, D, HW)
    T = min(4096, HW)
    NT = HW // T
    x, idx = pl.pallas_call(
        _vq_body,
        grid=(B, NT),
        in_specs=[
            pl.BlockSpec((1, D, T), lambda b, t: (b, 0, t)),
            pl.BlockSpec((K, D), lambda b, t: (0, 0)),
        ],
        out_specs=[
            pl.BlockSpec((1, T, D), lambda b, t: (b, t, 0)),
            pl.BlockSpec((1, 1, 1, T), lambda b, t: (b, t, 0, 0)),
        ],
        out_shape=[
            jax.ShapeDtypeStruct((B, HW, D), jnp.float32),
            jax.ShapeDtypeStruct((B, NT, 1, T), jnp.int32),
        ],
        scratch_shapes=[pltpu.VMEM((K, 1), jnp.float32)],
    )(z3, codebook)
    return x, idx.reshape(B, HW)
